# baseline (speedup 1.0000x reference)
import jax
import jax.numpy as jnp
from jax import lax
from jax.experimental import pallas as pl
from jax.experimental.pallas import tpu as pltpu
from jax.experimental.pallas import tpu_sc as plsc

N = 10000
E = 320000
DIN = 128
H = 32

NC = 2
NS = 16
LANES = 16
NW = NC * NS
PW = E // NW
C = 125
NCHUNK = PW // C
N_PAD = 10240
NPS = N_PAD // NS


def _relu(x):
    return jnp.maximum(x, 0.0)


def _norm_body(e_ref, o_ref):
    o_ref[...] = jnp.max(jnp.abs(e_ref[...]), keepdims=True).reshape(1, 1)


def _node0_body(nodes_ref, wne_ref, bne_ref, wes_ref, wer_ref,
                hn_ref, ps_ref, pr_ref):
    hn = _relu(jnp.dot(nodes_ref[...], wne_ref[...],
                       preferred_element_type=jnp.float32) + bne_ref[...])
    hn_ref[...] = hn
    ps_ref[...] = jnp.dot(hn, wes_ref[...], preferred_element_type=jnp.float32)
    pr_ref[...] = jnp.dot(hn, wer_ref[...], preferred_element_type=jnp.float32)


def _nodeupd_body(hn_ref, agg2_ref, wnh_ref, wna_ref, bn_ref, wes_ref, wer_ref,
                  hn_out, ps_ref, pr_ref):
    agg = agg2_ref[0, :N, :] + agg2_ref[1, :N, :]
    hn = _relu(jnp.dot(hn_ref[...], wnh_ref[...],
                       preferred_element_type=jnp.float32)
               + jnp.dot(agg, wna_ref[...], preferred_element_type=jnp.float32)
               + bn_ref[...])
    hn_out[...] = hn
    ps_ref[...] = jnp.dot(hn, wes_ref[...], preferred_element_type=jnp.float32)
    pr_ref[...] = jnp.dot(hn, wer_ref[...], preferred_element_type=jnp.float32)


def _q0_body(e4t_ref, norm_ref, s_ref, bee4_ref, bd0_ref, be04_ref, q_ref):
    inv = 1.0 / norm_ref[0, 0]
    pre = lax.dot_general(e4t_ref[...], s_ref[...], (((0,), (0,)), ((), ())),
                          preferred_element_type=jnp.float32)
    he = _relu(pre * inv + bee4_ref[...])
    q_ref[...] = jnp.dot(he, bd0_ref[...],
                         preferred_element_type=jnp.float32) + be04_ref[...]


def _q_body(m4_ref, bd_ref, be4_ref, q_ref):
    q_ref[...] = jnp.dot(m4_ref[...], bd_ref[...],
                         preferred_element_type=jnp.float32) + be4_ref[...]


def _dec_body(m4_ref, wedbd8_ref, d_ref):
    d_ref[...] = lax.dot_general(wedbd8_ref[...], m4_ref[...],
                                 (((1,), (1,)), ((), ())),
                                 preferred_element_type=jnp.float32)


def _out_body(dec_ref, e_ref, r_ref, s_ref, bed_ref, norm_ref, alpha_ref,
              o_ref):
    val = e_ref[...] + alpha_ref[0, 0] * norm_ref[0, 0] * (
        dec_ref[...] + bed_ref[0, 0])
    o_ref[...] = jnp.where(r_ref[...] >= s_ref[...], val, 0.0)


BE4 = 4000
BQ0 = 16000

_f32 = jnp.float32


def _blk(shape):
    return pl.BlockSpec(shape, lambda i: (i,) + (0,) * (len(shape) - 1))


def _rep(shape):
    return pl.BlockSpec(shape, lambda i: (0,) * len(shape))


def _make_sc_edge(with_agg):
    out_type = [jax.ShapeDtypeStruct((E, H), _f32)]
    scratch = [
        pltpu.VMEM((NCHUNK, C), jnp.int32),
        pltpu.VMEM((NCHUNK, C), jnp.int32),
        pltpu.VMEM((C, H), _f32),
        pltpu.VMEM((C, H), _f32),
        pltpu.VMEM((C, H), _f32),
        pltpu.VMEM((C, H), _f32),
        pltpu.VMEM((C, H), _f32),
        pltpu.VMEM((C, H), _f32),
        pltpu.VMEM((C, H), _f32),
        pltpu.VMEM((C, H), _f32),
    ] + [pltpu.SemaphoreType.DMA] * 8 + [
        pltpu.VMEM_SHARED((N, H), _f32),
    ]
    if with_agg:
        out_type.append(jax.ShapeDtypeStruct((NC, N_PAD, H), _f32))
        scratch.append(pltpu.VMEM((NPS, H), _f32))
        scratch.append(pltpu.VMEM_SHARED((N_PAD, H), _f32))

    mesh = plsc.VectorSubcoreMesh(core_axis_name="c", subcore_axis_name="s",
                                  num_cores=NC, num_subcores=NS)

    def body(ps_hbm, pr_hbm, q_hbm, snd_hbm, rcv_hbm, m_hbm, *rest):
        if with_agg:
            (agg_hbm, s_all, r_all, a0, a1, b0, b1, q0, q1, m0, m1,
             sa0, sa1, sb0, sb1, sq0, sq1, sm0, sm1,
             ps_sh, zbuf, agg_sh) = rest
        else:
            (s_all, r_all, a0, a1, b0, b1, q0, q1, m0, m1,
             sa0, sa1, sb0, sb1, sq0, sq1, sm0, sm1, ps_sh) = rest
        A, B, Q, M = (a0, a1), (b0, b1), (q0, q1), (m0, m1)
        SA, SB, SQ, SM = (sa0, sa1), (sb0, sb1), (sq0, sq1), (sm0, sm1)
        cid = lax.axis_index("c")
        sid = lax.axis_index("s")
        wid = sid * NC + cid
        base = wid * PW

        pltpu.sync_copy(snd_hbm.at[wid], s_all)
        pltpu.sync_copy(rcv_hbm.at[wid], r_all)

        nsl = N // NS
        psl = pl.ds(sid * nsl, nsl)
        pltpu.sync_copy(ps_hbm.at[psl], ps_sh.at[psl])
        plsc.subcore_barrier()

        if with_agg:
            def zrow(i, carry):
                zbuf[i, pl.ds(0, LANES)] = jnp.zeros((LANES,), _f32)
                zbuf[i, pl.ds(LANES, LANES)] = jnp.zeros((LANES,), _f32)
                return carry
            lax.fori_loop(0, NPS, zrow, 0)
            pltpu.sync_copy(zbuf, agg_sh.at[pl.ds(sid * NPS, NPS)])
            plsc.subcore_barrier()

        def fire(j, k):
            pltpu.async_copy(ps_sh.at[s_all.at[j]], A[k], SA[k])
            pltpu.async_copy(pr_hbm.at[r_all.at[j]], B[k], SB[k])
            pltpu.async_copy(q_hbm.at[pl.ds(base + j * C, C)], Q[k], SQ[k])

        def process(j, k, mwait):
            pltpu.make_async_copy(ps_sh.at[s_all.at[j]], A[k], SA[k]).wait()
            pltpu.make_async_copy(pr_hbm.at[r_all.at[j]], B[k], SB[k]).wait()
            pltpu.make_async_copy(q_hbm.at[pl.ds(base + j * C, C)], Q[k],
                                  SQ[k]).wait()
            if mwait:
                pltpu.make_async_copy(M[k], m_hbm.at[pl.ds(base, C)],
                                      SM[k]).wait()

            def row(c, carry):
                for h in range(H // LANES):
                    sl = pl.ds(h * LANES, LANES)
                    M[k][c, sl] = jnp.maximum(
                        Q[k][c, sl] + A[k][c, sl] + B[k][c, sl], 0.0)
                return carry
            lax.fori_loop(0, C, row, 0)

            pltpu.async_copy(M[k], m_hbm.at[pl.ds(base + j * C, C)], SM[k])
            if with_agg:
                pltpu.sync_copy(M[k], agg_sh.at[r_all.at[j]], add=True)

        fire(0, 0)
        fire(1, 1)
        process(0, 0, False)
        fire(2, 0)
        process(1, 1, False)
        fire(3, 1)

        def pair(t, carry):
            process(2 * t, 0, True)
            fire(2 * t + 2, 0)
            process(2 * t + 1, 1, True)
            fire(2 * t + 3, 1)
            return carry
        lax.fori_loop(1, NCHUNK // 2 - 1, pair, 0)

        process(NCHUNK - 2, 0, True)
        process(NCHUNK - 1, 1, True)
        pltpu.make_async_copy(M[0], m_hbm.at[pl.ds(base, C)], SM[0]).wait()
        pltpu.make_async_copy(M[1], m_hbm.at[pl.ds(base, C)], SM[1]).wait()

        if with_agg:
            plsc.subcore_barrier()
            pltpu.sync_copy(agg_sh.at[pl.ds(sid * NPS, NPS)],
                            agg_hbm.at[cid, pl.ds(sid * NPS, NPS)])

    return pl.kernel(body, out_type=tuple(out_type), mesh=mesh,
                     scratch_types=tuple(scratch),
                     compiler_params=pltpu.CompilerParams(
                         use_tc_tiling_on_sc=False))


_sc_edge_agg = _make_sc_edge(True)
_sc_edge_noagg = _make_sc_edge(False)


@jax.jit
def kernel(nodes, edges, receivers, senders, Wne, bne, Wee, bee, We, be, Wn,
           bn, Wed, bed, alpha):
    e2d = edges.reshape(E // 128, 128)
    r2d = receivers.reshape(E // 128, 128)
    s2d = senders.reshape(E // 128, 128)

    norm = pl.pallas_call(
        _norm_body,
        out_shape=jax.ShapeDtypeStruct((1, 1), _f32),
    )(e2d)

    eye4 = jnp.eye(4, dtype=_f32)
    bne2 = bne.reshape(1, H)
    bee4 = jnp.tile(bee.reshape(1, H), (1, 4))
    smat = jnp.kron(eye4, Wee)
    wedbd = jnp.kron(eye4, Wed)
    bed2 = bed.reshape(1, 1)
    alpha2 = alpha.reshape(1, 1)
    wes = [We[i, :H] for i in range(3)]
    wer = [We[i, H:2 * H] for i in range(3)]
    bd = [jnp.kron(eye4, We[i, 2 * H:]) for i in range(3)]
    be4 = [jnp.tile(be[i].reshape(1, H), (1, 4)) for i in range(3)]
    wnh = [Wn[i, :H] for i in range(2)]
    wna = [Wn[i, H:] for i in range(2)]
    bni = [bn[i].reshape(1, H) for i in range(2)]

    hn, ps, pr = pl.pallas_call(
        _node0_body,
        out_shape=[jax.ShapeDtypeStruct((N, H), _f32)] * 3,
    )(nodes, Wne, bne2, wes[0], wer[0])

    e4t = jnp.concatenate(
        [e2d.reshape(E // 4, 4).T,
         jnp.zeros((4, E // 4), _f32)], axis=0)
    smat8 = jnp.concatenate([smat, jnp.zeros((4, 128), _f32)], axis=0)
    q4 = pl.pallas_call(
        _q0_body,
        grid=(E // 4 // BQ0,),
        in_specs=[pl.BlockSpec((8, BQ0), lambda i: (0, i)), _rep((1, 1)),
                  _rep((8, 128)), _rep((1, 128)), _rep((128, 128)),
                  _rep((1, 128))],
        out_specs=_blk((BQ0, 128)),
        out_shape=jax.ShapeDtypeStruct((E // 4, 128), _f32),
    )(e4t, norm, smat8, bee4, bd[0], be4[0])

    snd3 = senders.reshape(NW, NCHUNK, C)
    rcv3 = receivers.reshape(NW, NCHUNK, C)

    m = None
    agg2 = None
    for i in range(3):
        if i > 0:
            hn, ps, pr = pl.pallas_call(
                _nodeupd_body,
                out_shape=[jax.ShapeDtypeStruct((N, H), _f32)] * 3,
            )(hn, agg2, wnh[i - 1], wna[i - 1], bni[i - 1], wes[i], wer[i])
            q4 = pl.pallas_call(
                _q_body,
                grid=(E // 4 // BE4,),
                in_specs=[_blk((BE4, 128)), _rep((128, 128)), _rep((1, 128))],
                out_specs=_blk((BE4, 128)),
                out_shape=jax.ShapeDtypeStruct((E // 4, 128), _f32),
            )(m4, bd[i], be4[i])
        q = q4.reshape(E, H)
        if i < 2:
            m, agg2 = _sc_edge_agg(ps, pr, q, snd3, rcv3)
        else:
            m = _sc_edge_noagg(ps, pr, q, snd3, rcv3)
            if isinstance(m, (tuple, list)):
                m = m[0]
        m4 = m.reshape(E // 4, 128)

    wedbd8 = jnp.concatenate(
        [wedbd.T, jnp.zeros((4, 128), _f32)], axis=0)
    dect = pl.pallas_call(
        _dec_body,
        grid=(E // 4 // BQ0,),
        in_specs=[_blk((BQ0, 128)), _rep((8, 128))],
        out_specs=pl.BlockSpec((8, BQ0), lambda i: (0, i)),
        out_shape=jax.ShapeDtypeStruct((8, E // 4), _f32),
    )(m4, wedbd8)
    dec2d = dect[:4].T.reshape(E // 128, 128)

    out2 = pl.pallas_call(
        _out_body,
        out_shape=jax.ShapeDtypeStruct((E // 128, 128), _f32),
    )(dec2d, e2d, r2d, s2d, bed2, norm, alpha2)
    return out2.reshape(E)


# speedup vs baseline: 1.0122x; 1.0122x over previous
import jax
import jax.numpy as jnp
from jax import lax
from jax.experimental import pallas as pl
from jax.experimental.pallas import tpu as pltpu
from jax.experimental.pallas import tpu_sc as plsc

N = 10000
E = 320000
DIN = 128
H = 32

NC = 2
NS = 16
LANES = 16
NW = NC * NS
PW = E // NW
C = 125
NCHUNK = PW // C
N_PAD = 10240
NPS = N_PAD // NS


def _relu(x):
    return jnp.maximum(x, 0.0)


def _norm_body(e_ref, o_ref):
    o_ref[...] = jnp.max(jnp.abs(e_ref[...]), keepdims=True).reshape(1, 1)


def _node0_body(nodes_ref, wne_ref, bne_ref, wes_ref, wer_ref,
                hn_ref, ps_ref, pr_ref):
    hn = _relu(jnp.dot(nodes_ref[...], wne_ref[...],
                       preferred_element_type=jnp.float32) + bne_ref[...])
    hn_ref[...] = hn
    ps_ref[...] = jnp.dot(hn, wes_ref[...], preferred_element_type=jnp.float32)
    pr_ref[...] = jnp.dot(hn, wer_ref[...], preferred_element_type=jnp.float32)


def _nodeupd_body(hn_ref, agg2_ref, wnh_ref, wna_ref, bn_ref, wes_ref, wer_ref,
                  hn_out, ps_ref, pr_ref):
    agg = agg2_ref[0, :N, :] + agg2_ref[1, :N, :]
    hn = _relu(jnp.dot(hn_ref[...], wnh_ref[...],
                       preferred_element_type=jnp.float32)
               + jnp.dot(agg, wna_ref[...], preferred_element_type=jnp.float32)
               + bn_ref[...])
    hn_out[...] = hn
    ps_ref[...] = jnp.dot(hn, wes_ref[...], preferred_element_type=jnp.float32)
    pr_ref[...] = jnp.dot(hn, wer_ref[...], preferred_element_type=jnp.float32)


def _q0_body(e4t_ref, norm_ref, s_ref, bee4_ref, bd0_ref, be04_ref, q_ref):
    inv = 1.0 / norm_ref[0, 0]
    pre = lax.dot_general(e4t_ref[...], s_ref[...], (((0,), (0,)), ((), ())),
                          preferred_element_type=jnp.float32)
    he = _relu(pre * inv + bee4_ref[...])
    q_ref[...] = jnp.dot(he, bd0_ref[...],
                         preferred_element_type=jnp.float32) + be04_ref[...]


def _q_body(m4_ref, bd_ref, be4_ref, q_ref):
    q_ref[...] = jnp.dot(m4_ref[...], bd_ref[...],
                         preferred_element_type=jnp.float32) + be4_ref[...]


def _dec_body(m4_ref, wedbd8_ref, d_ref):
    d_ref[...] = lax.dot_general(wedbd8_ref[...], m4_ref[...],
                                 (((1,), (1,)), ((), ())),
                                 preferred_element_type=jnp.float32)


def _out_body(dec_ref, e_ref, r_ref, s_ref, bed_ref, norm_ref, alpha_ref,
              o_ref):
    val = e_ref[...] + alpha_ref[0, 0] * norm_ref[0, 0] * (
        dec_ref[...] + bed_ref[0, 0])
    o_ref[...] = jnp.where(r_ref[...] >= s_ref[...], val, 0.0)


BE4 = 8000
BQ0 = 16000

_f32 = jnp.float32


def _blk(shape):
    return pl.BlockSpec(shape, lambda i: (i,) + (0,) * (len(shape) - 1))


def _rep(shape):
    return pl.BlockSpec(shape, lambda i: (0,) * len(shape))


def _make_sc_edge(with_agg):
    out_type = [jax.ShapeDtypeStruct((E, H), _f32)]
    scratch = [
        pltpu.VMEM((NCHUNK, C), jnp.int32),
        pltpu.VMEM((NCHUNK, C), jnp.int32),
        pltpu.VMEM((C, H), _f32),
        pltpu.VMEM((C, H), _f32),
        pltpu.VMEM((C, H), _f32),
        pltpu.VMEM((C, H), _f32),
        pltpu.VMEM((C, H), _f32),
        pltpu.VMEM((C, H), _f32),
        pltpu.VMEM((C, H), _f32),
        pltpu.VMEM((C, H), _f32),
    ] + [pltpu.SemaphoreType.DMA] * 8 + [
        pltpu.VMEM_SHARED((N, H), _f32),
    ]
    if with_agg:
        out_type.append(jax.ShapeDtypeStruct((NC, N_PAD, H), _f32))
        scratch.append(pltpu.VMEM((NPS, H), _f32))
        scratch.append(pltpu.VMEM_SHARED((N_PAD, H), _f32))

    mesh = plsc.VectorSubcoreMesh(core_axis_name="c", subcore_axis_name="s",
                                  num_cores=NC, num_subcores=NS)

    def body(ps_hbm, pr_hbm, q_hbm, snd_hbm, rcv_hbm, m_hbm, *rest):
        if with_agg:
            (agg_hbm, s_all, r_all, a0, a1, b0, b1, q0, q1, m0, m1,
             sa0, sa1, sb0, sb1, sq0, sq1, sm0, sm1,
             ps_sh, zbuf, agg_sh) = rest
        else:
            (s_all, r_all, a0, a1, b0, b1, q0, q1, m0, m1,
             sa0, sa1, sb0, sb1, sq0, sq1, sm0, sm1, ps_sh) = rest
        A, B, Q, M = (a0, a1), (b0, b1), (q0, q1), (m0, m1)
        SA, SB, SQ, SM = (sa0, sa1), (sb0, sb1), (sq0, sq1), (sm0, sm1)
        cid = lax.axis_index("c")
        sid = lax.axis_index("s")
        wid = sid * NC + cid
        base = wid * PW

        pltpu.sync_copy(snd_hbm.at[wid], s_all)
        pltpu.sync_copy(rcv_hbm.at[wid], r_all)

        nsl = N // NS
        psl = pl.ds(sid * nsl, nsl)
        pltpu.sync_copy(ps_hbm.at[psl], ps_sh.at[psl])
        plsc.subcore_barrier()

        if with_agg:
            def zrow(i, carry):
                zbuf[i, pl.ds(0, LANES)] = jnp.zeros((LANES,), _f32)
                zbuf[i, pl.ds(LANES, LANES)] = jnp.zeros((LANES,), _f32)
                return carry
            lax.fori_loop(0, NPS, zrow, 0)
            pltpu.sync_copy(zbuf, agg_sh.at[pl.ds(sid * NPS, NPS)])
            plsc.subcore_barrier()

        def fire(j, k):
            pltpu.async_copy(ps_sh.at[s_all.at[j]], A[k], SA[k])
            pltpu.async_copy(pr_hbm.at[r_all.at[j]], B[k], SB[k])
            pltpu.async_copy(q_hbm.at[pl.ds(base + j * C, C)], Q[k], SQ[k])

        def process(j, k, mwait):
            pltpu.make_async_copy(ps_sh.at[s_all.at[j]], A[k], SA[k]).wait()
            pltpu.make_async_copy(pr_hbm.at[r_all.at[j]], B[k], SB[k]).wait()
            pltpu.make_async_copy(q_hbm.at[pl.ds(base + j * C, C)], Q[k],
                                  SQ[k]).wait()
            if mwait:
                pltpu.make_async_copy(M[k], m_hbm.at[pl.ds(base, C)],
                                      SM[k]).wait()

            def row(c, carry):
                for h in range(H // LANES):
                    sl = pl.ds(h * LANES, LANES)
                    M[k][c, sl] = jnp.maximum(
                        Q[k][c, sl] + A[k][c, sl] + B[k][c, sl], 0.0)
                return carry
            lax.fori_loop(0, C, row, 0)

            pltpu.async_copy(M[k], m_hbm.at[pl.ds(base + j * C, C)], SM[k])
            if with_agg:
                pltpu.sync_copy(M[k], agg_sh.at[r_all.at[j]], add=True)

        fire(0, 0)
        fire(1, 1)
        process(0, 0, False)
        fire(2, 0)
        process(1, 1, False)
        fire(3, 1)

        def pair(t, carry):
            process(2 * t, 0, True)
            fire(2 * t + 2, 0)
            process(2 * t + 1, 1, True)
            fire(2 * t + 3, 1)
            return carry
        lax.fori_loop(1, NCHUNK // 2 - 1, pair, 0)

        process(NCHUNK - 2, 0, True)
        process(NCHUNK - 1, 1, True)
        pltpu.make_async_copy(M[0], m_hbm.at[pl.ds(base, C)], SM[0]).wait()
        pltpu.make_async_copy(M[1], m_hbm.at[pl.ds(base, C)], SM[1]).wait()

        if with_agg:
            plsc.subcore_barrier()
            pltpu.sync_copy(agg_sh.at[pl.ds(sid * NPS, NPS)],
                            agg_hbm.at[cid, pl.ds(sid * NPS, NPS)])

    return pl.kernel(body, out_type=tuple(out_type), mesh=mesh,
                     scratch_types=tuple(scratch),
                     compiler_params=pltpu.CompilerParams(
                         use_tc_tiling_on_sc=False))


_sc_edge_agg = _make_sc_edge(True)
_sc_edge_noagg = _make_sc_edge(False)


@jax.jit
def kernel(nodes, edges, receivers, senders, Wne, bne, Wee, bee, We, be, Wn,
           bn, Wed, bed, alpha):
    e2d = edges.reshape(E // 128, 128)
    r2d = receivers.reshape(E // 128, 128)
    s2d = senders.reshape(E // 128, 128)

    norm = pl.pallas_call(
        _norm_body,
        out_shape=jax.ShapeDtypeStruct((1, 1), _f32),
    )(e2d)

    eye4 = jnp.eye(4, dtype=_f32)
    bne2 = bne.reshape(1, H)
    bee4 = jnp.tile(bee.reshape(1, H), (1, 4))
    smat = jnp.kron(eye4, Wee)
    wedbd = jnp.kron(eye4, Wed)
    bed2 = bed.reshape(1, 1)
    alpha2 = alpha.reshape(1, 1)
    wes = [We[i, :H] for i in range(3)]
    wer = [We[i, H:2 * H] for i in range(3)]
    bd = [jnp.kron(eye4, We[i, 2 * H:]) for i in range(3)]
    be4 = [jnp.tile(be[i].reshape(1, H), (1, 4)) for i in range(3)]
    wnh = [Wn[i, :H] for i in range(2)]
    wna = [Wn[i, H:] for i in range(2)]
    bni = [bn[i].reshape(1, H) for i in range(2)]

    hn, ps, pr = pl.pallas_call(
        _node0_body,
        out_shape=[jax.ShapeDtypeStruct((N, H), _f32)] * 3,
    )(nodes, Wne, bne2, wes[0], wer[0])

    e4t = jnp.concatenate(
        [e2d.reshape(E // 4, 4).T,
         jnp.zeros((4, E // 4), _f32)], axis=0)
    smat8 = jnp.concatenate([smat, jnp.zeros((4, 128), _f32)], axis=0)
    q4 = pl.pallas_call(
        _q0_body,
        grid=(E // 4 // BQ0,),
        in_specs=[pl.BlockSpec((8, BQ0), lambda i: (0, i)), _rep((1, 1)),
                  _rep((8, 128)), _rep((1, 128)), _rep((128, 128)),
                  _rep((1, 128))],
        out_specs=_blk((BQ0, 128)),
        out_shape=jax.ShapeDtypeStruct((E // 4, 128), _f32),
    )(e4t, norm, smat8, bee4, bd[0], be4[0])

    snd3 = senders.reshape(NW, NCHUNK, C)
    rcv3 = receivers.reshape(NW, NCHUNK, C)

    m = None
    agg2 = None
    for i in range(3):
        if i > 0:
            hn, ps, pr = pl.pallas_call(
                _nodeupd_body,
                out_shape=[jax.ShapeDtypeStruct((N, H), _f32)] * 3,
            )(hn, agg2, wnh[i - 1], wna[i - 1], bni[i - 1], wes[i], wer[i])
            q4 = pl.pallas_call(
                _q_body,
                grid=(E // 4 // BE4,),
                in_specs=[_blk((BE4, 128)), _rep((128, 128)), _rep((1, 128))],
                out_specs=_blk((BE4, 128)),
                out_shape=jax.ShapeDtypeStruct((E // 4, 128), _f32),
            )(m4, bd[i], be4[i])
        q = q4.reshape(E, H)
        if i < 2:
            m, agg2 = _sc_edge_agg(ps, pr, q, snd3, rcv3)
        else:
            m = _sc_edge_noagg(ps, pr, q, snd3, rcv3)
            if isinstance(m, (tuple, list)):
                m = m[0]
        m4 = m.reshape(E // 4, 128)

    wedbd8 = jnp.concatenate(
        [wedbd.T, jnp.zeros((4, 128), _f32)], axis=0)
    dect = pl.pallas_call(
        _dec_body,
        grid=(E // 4 // BQ0,),
        in_specs=[_blk((BQ0, 128)), _rep((8, 128))],
        out_specs=pl.BlockSpec((8, BQ0), lambda i: (0, i)),
        out_shape=jax.ShapeDtypeStruct((8, E // 4), _f32),
    )(m4, wedbd8)
    dec2d = dect[:4].T.reshape(E // 128, 128)

    out2 = pl.pallas_call(
        _out_body,
        out_shape=jax.ShapeDtypeStruct((E // 128, 128), _f32),
    )(dec2d, e2d, r2d, s2d, bed2, norm, alpha2)
    return out2.reshape(E)
